# baseline (speedup 1.0000x reference)
import jax
import jax.numpy as jnp
from jax import lax
from jax.experimental import pallas as pl
from jax.experimental.pallas import tpu as pltpu
from jax.experimental.pallas import tpu_sc as plsc

BATCH = 4
EMB_DIM = 1024
SEQ_LEN = 8192

NC = 2
NS = 16
LANES = 16
NW = NC * NS

D_GRP = 128
N_DG = EMB_DIM // D_GRP
N_SG = NW // N_DG
S_PER_W = SEQ_LEN // N_SG
S_B = 128
NCHUNK = S_PER_W // S_B
D_SUB = 32
N_SUB = D_GRP // D_SUB


def _sc_body(x_hbm, pe_hbm, out_hbm, pebuf, xbuf, outbuf,
             sem_pe, sem_x, sem_out):
    wid = lax.axis_index("s") * NC + lax.axis_index("c")
    dg = wid // N_SG
    sg = wid % N_SG
    d0 = dg * D_GRP
    s_base = sg * S_PER_W
    iota = lax.iota(jnp.int32, LANES)
    idx_s_list = [iota + (sc * LANES) for sc in range(S_B // LANES)]

    def pe_copy(c, buf):
        return pltpu.make_async_copy(
            pe_hbm.at[pl.ds(s_base + c * S_B, S_B), pl.ds(d0, D_GRP)],
            pebuf.at[buf, :, pl.ds(0, D_GRP)], sem_pe)

    def x_copies(c, sub, p):
        s0 = s_base + c * S_B
        dsub0 = d0 + sub * D_SUB
        return [pltpu.make_async_copy(
            x_hbm.at[:, pl.ds(dsub0, D_SUB), pl.ds(s0, S_B)],
            xbuf.at[p], sem_x)]

    def out_copies(c, sub, p):
        s0 = s_base + c * S_B
        dsub0 = d0 + sub * D_SUB
        return [pltpu.make_async_copy(
            outbuf.at[p],
            out_hbm.at[:, pl.ds(dsub0, D_SUB), pl.ds(s0, S_B)],
            sem_out)]

    pe_copy(0, 0).start()
    for cp_ in x_copies(0, 0, 0):
        cp_.start()

    def cc_body(cc, carry):
        for cpar in range(2):
            c = cc * 2 + cpar
            for sub in range(N_SUB):
                p = sub % 2
                r = c * N_SUB + sub

                r2 = r - 2
                c2 = r2 // N_SUB
                sub2 = lax.rem(r2, N_SUB)

                @pl.when(r >= 2)
                def _():
                    s0_2 = s_base + c2 * S_B
                    dsub0_2 = d0 + sub2 * D_SUB
                    pltpu.make_async_copy(
                        outbuf.at[p],
                        out_hbm.at[:, pl.ds(dsub0_2, D_SUB),
                                   pl.ds(s0_2, S_B)],
                        sem_out).wait()

                if sub == 0:
                    @pl.when(c + 1 < NCHUNK)
                    def _():
                        pe_copy(c + 1, 1 - cpar).start()
                    pe_copy(c, cpar).wait()

                if sub < N_SUB - 1:
                    for cp_ in x_copies(c, sub + 1, 1 - p):
                        cp_.start()
                else:
                    @pl.when(c + 1 < NCHUNK)
                    def _():
                        for cp_ in x_copies(c + 1, 0, 1 - p):
                            cp_.start()

                for cp_ in x_copies(c, sub, p):
                    cp_.wait()

                peref = pebuf.at[cpar]

                idx_dbase = jnp.full((LANES,), sub * D_SUB, jnp.int32)

                @plsc.parallel_loop(0, D_SUB, 1, unroll=8)
                def dl_body(dl, _p=p, _idx_dbase=idx_dbase):
                    idx_d = _idx_dbase + dl
                    for sc in range(S_B // LANES):
                        pev = plsc.load_gather(peref, [idx_s_list[sc], idx_d])
                        for b in range(BATCH):
                            xv = xbuf[_p, b, dl, pl.ds(sc * LANES, LANES)]
                            outbuf[_p, b, dl, pl.ds(sc * LANES, LANES)] = (
                                xv + pev)

                for cp_ in out_copies(c, sub, p):
                    cp_.start()
        return carry

    lax.fori_loop(0, NCHUNK // 2, cc_body, 0)

    for r in (NCHUNK * N_SUB - 2, NCHUNK * N_SUB - 1):
        c = r // N_SUB
        sub = r % N_SUB
        for cp_ in out_copies(c, sub, sub % 2):
            cp_.wait()


def kernel(x, pe_table):
    mesh = plsc.VectorSubcoreMesh(core_axis_name="c", subcore_axis_name="s")
    k = pl.kernel(
        _sc_body,
        out_type=jax.ShapeDtypeStruct((BATCH, EMB_DIM, SEQ_LEN), jnp.float32),
        mesh=mesh,
        compiler_params=pltpu.CompilerParams(needs_layout_passes=False),
        scratch_types=[
            pltpu.VMEM((2, S_B, D_GRP + 1), jnp.float32),
            pltpu.VMEM((2, BATCH, D_SUB, S_B), jnp.float32),
            pltpu.VMEM((2, BATCH, D_SUB, S_B), jnp.float32),
            pltpu.SemaphoreType.DMA,
            pltpu.SemaphoreType.DMA,
            pltpu.SemaphoreType.DMA,
        ],
    )
    return k(x, pe_table)


# speedup vs baseline: 2.1991x; 2.1991x over previous
import jax
import jax.numpy as jnp
from jax.experimental import pallas as pl

BATCH = 4
EMB_DIM = 1024
SEQ_LEN = 8192

D_BLK = 1024
S_BLK = 256


def _body(x_ref, pe_ref, o_ref):
    pe_t = jnp.transpose(pe_ref[...], (1, 0))
    o_ref[...] = x_ref[...] + pe_t[None, :, :]


def kernel(x, pe_table):
    grid = (EMB_DIM // D_BLK, SEQ_LEN // S_BLK)
    return pl.pallas_call(
        _body,
        grid=grid,
        in_specs=[
            pl.BlockSpec((BATCH, D_BLK, S_BLK), lambda i, j: (0, i, j)),
            pl.BlockSpec((S_BLK, D_BLK), lambda i, j: (j, i)),
        ],
        out_specs=pl.BlockSpec((BATCH, D_BLK, S_BLK), lambda i, j: (0, i, j)),
        out_shape=jax.ShapeDtypeStruct((BATCH, EMB_DIM, SEQ_LEN), jnp.float32),
    )(x, pe_table)


# speedup vs baseline: 2.2177x; 1.0085x over previous
import jax
import jax.numpy as jnp
from jax.experimental import pallas as pl

BATCH = 4
EMB_DIM = 1024
SEQ_LEN = 8192

D_BLK = 1024
S_BLK = 512


def _body(x_ref, pe_ref, o_ref):
    pe_t = jnp.transpose(pe_ref[...], (1, 0))
    o_ref[...] = x_ref[...] + pe_t[None, :, :]


def kernel(x, pe_table):
    grid = (EMB_DIM // D_BLK, SEQ_LEN // S_BLK)
    return pl.pallas_call(
        _body,
        grid=grid,
        in_specs=[
            pl.BlockSpec((BATCH, D_BLK, S_BLK), lambda i, j: (0, i, j)),
            pl.BlockSpec((S_BLK, D_BLK), lambda i, j: (j, i)),
        ],
        out_specs=pl.BlockSpec((BATCH, D_BLK, S_BLK), lambda i, j: (0, i, j)),
        out_shape=jax.ShapeDtypeStruct((BATCH, EMB_DIM, SEQ_LEN), jnp.float32),
    )(x, pe_table)
